# speedup vs baseline: 2.9702x; 2.9702x over previous
import functools

import jax
import jax.numpy as jnp
from jax import lax
from jax.experimental import pallas as pl
from jax.experimental.pallas import tpu as pltpu
from jax.experimental.pallas import tpu_sc as plsc

D = 128
NC = 2
NS = 16
NW = NC * NS
CHUNK = 128


def _embed_body(n_chunks, table_hbm, idx_hbm, out_hbm, idx_v, rows_v, sem):
    c = lax.axis_index("c")
    s = lax.axis_index("s")
    wid = s * NC + c
    b_per_w = n_chunks * CHUNK
    base = wid * b_per_w
    pltpu.sync_copy(idx_hbm.at[wid], idx_v)

    def step(j, carry):
        pltpu.async_copy(table_hbm.at[idx_v.at[j]], rows_v, sem).wait()
        pltpu.sync_copy(rows_v, out_hbm.at[pl.ds(base + j * CHUNK, CHUNK)])
        return carry

    lax.fori_loop(0, n_chunks, step, 0)


def kernel(x, table):
    b, h = x.shape
    n_total = b * h
    assert n_total % (NW * CHUNK) == 0
    n_chunks = n_total // (NW * CHUNK)
    idx = x.reshape(NW, n_chunks, CHUNK).astype(jnp.int32)

    run = pl.kernel(
        functools.partial(_embed_body, n_chunks),
        out_type=jax.ShapeDtypeStruct((n_total, D), table.dtype),
        mesh=plsc.VectorSubcoreMesh(core_axis_name="c", subcore_axis_name="s"),
        scratch_types=[
            pltpu.VMEM((n_chunks, CHUNK), jnp.int32),
            pltpu.VMEM((CHUNK, D), jnp.float32),
            pltpu.SemaphoreType.DMA,
        ],
    )
    out = run(table, idx)
    return out.reshape(b, h, D)


# speedup vs baseline: 3.3144x; 1.1159x over previous
import functools

import jax
import jax.numpy as jnp
from jax import lax
from jax.experimental import pallas as pl
from jax.experimental.pallas import tpu as pltpu
from jax.experimental.pallas import tpu_sc as plsc

D = 128
NC = 2
NS = 16
NW = NC * NS
CHUNK = 128
NBUF = 5


def _embed_body(n_chunks, table_hbm, idx_hbm, out_hbm, idx_v, rows_v, *sems):
    gsem = sems[:NBUF]
    ssem = sems[NBUF:]
    n_super = n_chunks // NBUF
    c = lax.axis_index("c")
    s = lax.axis_index("s")
    wid = s * NC + c
    b_per_w = n_chunks * CHUNK
    base = wid * b_per_w
    pltpu.sync_copy(idx_hbm.at[wid], idx_v)

    def gather(j, b):
        return pltpu.make_async_copy(
            table_hbm.at[idx_v.at[j]], rows_v.at[b], gsem[b])

    def store(j, b):
        return pltpu.make_async_copy(
            rows_v.at[b], out_hbm.at[pl.ds(base + j * CHUNK, CHUNK)], ssem[b])

    for b in range(NBUF):
        gather(b, b).start()

    def gen(g, carry):
        for b in range(NBUF):
            j = g * NBUF + b
            gather(j, b).wait()
            store(j, b).start()
        for b in range(NBUF):
            j = g * NBUF + b
            store(j, b).wait()
            gather(j + NBUF, b).start()
        return carry

    lax.fori_loop(0, n_super - 1, gen, 0)

    for b in range(NBUF):
        j = (n_super - 1) * NBUF + b
        gather(j, b).wait()
        store(j, b).start()
    for b in range(NBUF):
        j = (n_super - 1) * NBUF + b
        store(j, b).wait()


def kernel(x, table):
    b, h = x.shape
    n_total = b * h
    assert n_total % (NW * CHUNK) == 0
    n_chunks = n_total // (NW * CHUNK)
    assert n_chunks % NBUF == 0
    idx = x.reshape(NW, n_chunks, CHUNK).astype(jnp.int32)

    run = pl.kernel(
        functools.partial(_embed_body, n_chunks),
        out_type=jax.ShapeDtypeStruct((n_total, D), table.dtype),
        mesh=plsc.VectorSubcoreMesh(core_axis_name="c", subcore_axis_name="s"),
        scratch_types=[
            pltpu.VMEM((n_chunks, CHUNK), jnp.int32),
            pltpu.VMEM((NBUF, CHUNK, D), jnp.float32),
        ] + [pltpu.SemaphoreType.DMA] * (2 * NBUF),
    )
    out = run(table, idx)
    return out.reshape(b, h, D)


# speedup vs baseline: 5.9066x; 1.7821x over previous
import functools

import jax
import jax.numpy as jnp
from jax import lax
from jax.experimental import pallas as pl
from jax.experimental.pallas import tpu as pltpu
from jax.experimental.pallas import tpu_sc as plsc

D = 128
NC = 2
NS = 16
NW = NC * NS
BPC = 2
NBUF = 8


def _embed_body(n_chunks, hist, table_hbm, idx_hbm, out_hbm, idx_v, rows_v,
                *sems):
    gsem = sems[:NBUF]
    ssem = sems[NBUF:]
    n_super = n_chunks // NBUF
    rows_per_chunk = BPC * hist
    c = lax.axis_index("c")
    s = lax.axis_index("s")
    wid = s * NC + c
    batch_base = wid * (n_chunks * BPC)
    pltpu.sync_copy(idx_hbm.at[wid], idx_v)

    def gather(j, b):
        return pltpu.make_async_copy(
            table_hbm.at[idx_v.at[j, pl.ds(0, rows_per_chunk)]],
            rows_v.at[b], gsem[b])

    def stores(j, b):
        return [
            pltpu.make_async_copy(
                rows_v.at[b, pl.ds(k * hist, hist)],
                out_hbm.at[batch_base + j * BPC + k], ssem[b])
            for k in range(BPC)
        ]

    for b in range(NBUF):
        gather(b, b).start()

    def gen(g, carry):
        for b in range(NBUF):
            j = g * NBUF + b
            gather(j, b).wait()
            for st in stores(j, b):
                st.start()
        for b in range(NBUF):
            j = g * NBUF + b
            for st in stores(j, b):
                st.wait()
            gather(j + NBUF, b).start()
        return carry

    lax.fori_loop(0, n_super - 1, gen, 0)

    for b in range(NBUF):
        j = (n_super - 1) * NBUF + b
        gather(j, b).wait()
        for st in stores(j, b):
            st.start()
    for b in range(NBUF):
        j = (n_super - 1) * NBUF + b
        for st in stores(j, b):
            st.wait()


def kernel(x, table):
    bsz, hist = x.shape
    assert bsz % (NW * BPC) == 0
    n_chunks = bsz // (NW * BPC)
    rows_per_chunk = BPC * hist
    assert rows_per_chunk <= 128
    idx = x.reshape(NW, n_chunks, rows_per_chunk).astype(jnp.int32)
    idx = jnp.pad(idx, ((0, 0), (0, 0), (0, 128 - rows_per_chunk)))

    run = pl.kernel(
        functools.partial(_embed_body, n_chunks, hist),
        out_type=jax.ShapeDtypeStruct((bsz, hist, D), table.dtype),
        mesh=plsc.VectorSubcoreMesh(core_axis_name="c", subcore_axis_name="s"),
        scratch_types=[
            pltpu.VMEM((n_chunks, 128), jnp.int32),
            pltpu.VMEM((NBUF, rows_per_chunk, D), jnp.float32),
        ] + [pltpu.SemaphoreType.DMA] * (2 * NBUF),
    )
    return run(table, idx)


# speedup vs baseline: 10.1919x; 1.7255x over previous
import functools

import jax
import jax.numpy as jnp
from jax import lax
from jax.experimental import pallas as pl
from jax.experimental.pallas import tpu as pltpu
from jax.experimental.pallas import tpu_sc as plsc

D = 128
NC = 2
NS = 16
NW = NC * NS
CHUNK = 64
NBUF = 10


def _embed_body(n_chunks, table_hbm, idx_hbm, out_hbm, idx_v, rows_v, *sems):
    gsem = sems[:NBUF]
    ssem = sems[NBUF:]
    n_super = n_chunks // NBUF
    c = lax.axis_index("c")
    s = lax.axis_index("s")
    wid = s * NC + c
    base = wid * (n_chunks * CHUNK)
    pltpu.sync_copy(idx_hbm.at[wid], idx_v)

    def gather(j, b):
        return pltpu.make_async_copy(
            table_hbm.at[idx_v.at[j]], rows_v.at[b], gsem[b])

    def store(j, b):
        return pltpu.make_async_copy(
            rows_v.at[b], out_hbm.at[pl.ds(base + j * CHUNK, CHUNK)], ssem[b])

    for b in range(NBUF):
        gather(b, b).start()

    def gen(g, carry):
        for b in range(NBUF):
            j = g * NBUF + b
            gather(j, b).wait()
            store(j, b).start()
        for b in range(NBUF):
            j = g * NBUF + b
            store(j, b).wait()
            gather(j + NBUF, b).start()
        return carry

    lax.fori_loop(0, n_super - 1, gen, 0)

    for b in range(NBUF):
        j = (n_super - 1) * NBUF + b
        gather(j, b).wait()
        store(j, b).start()
    for b in range(NBUF):
        j = (n_super - 1) * NBUF + b
        store(j, b).wait()


def kernel(x, table):
    bsz, hist = x.shape
    n_total = bsz * hist
    assert n_total % (NW * CHUNK) == 0
    n_chunks = n_total // (NW * CHUNK)
    assert n_chunks % NBUF == 0
    idx = jnp.transpose(x).reshape(NW, n_chunks, CHUNK).astype(jnp.int32)

    run = pl.kernel(
        functools.partial(_embed_body, n_chunks),
        out_type=jax.ShapeDtypeStruct((n_total, D), table.dtype),
        mesh=plsc.VectorSubcoreMesh(core_axis_name="c", subcore_axis_name="s"),
        scratch_types=[
            pltpu.VMEM((n_chunks, CHUNK), jnp.int32),
            pltpu.VMEM((NBUF, CHUNK, D), jnp.float32),
        ] + [pltpu.SemaphoreType.DMA] * (2 * NBUF),
    )
    out = run(table, idx)
    return jnp.transpose(out.reshape(hist, bsz, D), (1, 0, 2))
